# baseline (speedup 1.0000x reference)
import functools

import jax
import jax.numpy as jnp
from jax import lax
from jax.experimental import pallas as pl
from jax.experimental.pallas import tpu as pltpu
from jax.experimental.pallas import tpu_sc as plsc

N = 10000
K = 32
NKP = 15
C = 128
EXT = 1.2
NK = N * K

NC = 2
NS = 16
NW = NC * NS
NKPAD = 327680
CH = 128
NCHUNKS = NKPAD // CH
N0 = 126
N1 = NCHUNKS // NS - N0
SPLIT0 = N0 * NS
NBUF = 3
NPT = 10240
XYZF = 3 * NPT
GS = 8
CROWS = NKPAD // (GS * K) * 8

P = 200
PK = P * K
GRID = N // P
NPAD = 10240


def _sc_gather(feats, xyzf, idx):
  mesh = plsc.VectorSubcoreMesh(core_axis_name="c", subcore_axis_name="s")

  @functools.partial(
      pl.kernel,
      out_type=[
          jax.ShapeDtypeStruct((NKPAD, C), jnp.float32),
          jax.ShapeDtypeStruct((CROWS, 256), jnp.float32),
      ],
      mesh=mesh,
      scratch_types=[
          pltpu.VMEM((XYZF,), jnp.float32),
          pltpu.VMEM((NBUF, CH), jnp.int32),
          pltpu.VMEM((NBUF, CH, C), jnp.float32),
          pltpu.VMEM((NBUF, 8, 128), jnp.float32),
          pltpu.SemaphoreType.DMA((NBUF,)),
          pltpu.SemaphoreType.DMA((NBUF,)),
          pltpu.SemaphoreType.DMA((NBUF,)),
          pltpu.SemaphoreType.DMA((NBUF,)),
      ],
      compiler_params=pltpu.CompilerParams(needs_layout_passes=False),
  )
  def sc_kernel(feats_hbm, xyz_hbm, idx_hbm, g_hbm, cen_hbm,
                xyz_t, idx_v, fbuf, cbuf, sem_i, sem_g, sem_fw, sem_cw):
    cid = lax.axis_index("c")
    sid = lax.axis_index("s")
    pltpu.sync_copy(xyz_hbm, xyz_t)
    lanes16 = lax.iota(jnp.int32, 16)
    zero16 = jnp.zeros((16,), jnp.float32)

    def initrow(i, carry):
      b = i // 64
      r = (i // 8) % 8
      s = i % 8
      cbuf[b, r, pl.ds(s * 16, 16)] = zero16
      return carry

    lax.fori_loop(0, NBUF * 64, initrow, 0)

    def pipeline(start, iters):
      def chunk_base(i):
        return pl.multiple_of((start + i) * CH, CH)

      def idx_load(i):
        s = i % NBUF
        pltpu.async_copy(idx_hbm.at[pl.ds(chunk_base(i), CH)],
                         idx_v.at[s], sem_i.at[s])

      def wait_idx(i):
        s = i % NBUF
        pltpu.make_async_copy(idx_hbm.at[pl.ds(chunk_base(i), CH)],
                              idx_v.at[s], sem_i.at[s]).wait()

      def feats_out(i):
        s = i % NBUF
        pltpu.make_async_copy(feats_hbm.at[idx_v.at[s]],
                              fbuf.at[s], sem_g.at[s]).wait()
        pltpu.async_copy(fbuf.at[s], g_hbm.at[pl.ds(chunk_base(i), CH)],
                         sem_fw.at[s])

      def wait_feats_out(i):
        s = i % NBUF
        pltpu.make_async_copy(fbuf.at[s], g_hbm.at[pl.ds(chunk_base(i), CH)],
                              sem_fw.at[s]).wait()

      def _cen_dst(i):
        base = chunk_base(i)
        row0 = pl.multiple_of((base // 256) * 8, 8)
        l0 = pl.multiple_of(lax.bitwise_and(base // 128, 1) * 128, 128)
        return cen_hbm.at[pl.ds(row0, 8), pl.ds(l0, 128)]

      def cen_out(i):
        s = i % NBUF
        pltpu.async_copy(cbuf.at[s], _cen_dst(i), sem_cw.at[s])

      def wait_cen_out(i):
        s = i % NBUF
        pltpu.make_async_copy(cbuf.at[s], _cen_dst(i), sem_cw.at[s]).wait()

      idx_load(0)
      idx_load(1)

      def body(i, carry):
        s = i % NBUF
        wait_idx(i)
        @pl.when(i >= NBUF)
        def _():
          wait_feats_out(i - NBUF)
        pltpu.async_copy(feats_hbm.at[idx_v.at[s]], fbuf.at[s], sem_g.at[s])
        @pl.when(i >= 1)
        def _():
          feats_out(i - 1)
        @pl.when(i + 2 < iters)
        def _():
          idx_load(i + 2)
        @pl.when(i >= NBUF)
        def _():
          wait_cen_out(i - NBUF)
        base = chunk_base(i)
        for g in range(CH // 16):
          iv = idx_v[s, pl.ds(16 * g, 16)]
          i3 = iv * 3
          r16 = base + 16 * g + lanes16
          q3 = lax.shift_right_logical(r16, 5) * 3
          cx = plsc.load_gather(xyz_t, [i3]) - plsc.load_gather(xyz_t, [q3])
          cy = (plsc.load_gather(xyz_t, [i3 + 1])
                - plsc.load_gather(xyz_t, [q3 + 1]))
          cz = (plsc.load_gather(xyz_t, [i3 + 2])
                - plsc.load_gather(xyz_t, [q3 + 2]))
          cbuf[s, 0, pl.ds(16 * g, 16)] = cx
          cbuf[s, 1, pl.ds(16 * g, 16)] = cy
          cbuf[s, 2, pl.ds(16 * g, 16)] = cz
        cen_out(i)
        return carry

      lax.fori_loop(0, iters, body, 0)
      feats_out(iters - 1)
      wait_feats_out(iters - 3)
      wait_feats_out(iters - 2)
      wait_feats_out(iters - 1)
      wait_cen_out(iters - 3)
      wait_cen_out(iters - 2)
      wait_cen_out(iters - 1)

    @pl.when(cid == 0)
    def _():
      pipeline(sid * N0, N0)

    @pl.when(cid != 0)
    def _():
      pipeline(SPLIT0 + sid * N1, N1)

  return sc_kernel(feats, xyzf, idx)


NGB = P // GS


def _tc_main_body(g_ref, cen_ref, kpa_ref, m2_ref, wcat_ref, y_ref, st_ref,
                  wfs_ref):
  i = pl.program_id(0)
  kpx = kpa_ref[0:16, :]
  kpy = kpa_ref[16:32, :]
  kpz = kpa_ref[32:48, :]
  kpq = kpa_ref[48:64, :]
  m2 = m2_ref[...]
  cen = cen_ref[...]
  g = g_ref[...]
  for gb in range(NGB):
    cx = cen[8 * gb + 0:8 * gb + 1, :]
    cy = cen[8 * gb + 1:8 * gb + 2, :]
    cz = cen[8 * gb + 2:8 * gb + 3, :]
    colsq = cx * cx + cy * cy + cz * cz
    d2 = kpq + colsq - 2.0 * (kpx * cx + kpy * cy + kpz * cz)
    wg = jnp.maximum(1.0 - jnp.sqrt(jnp.maximum(d2, 0.0)) * (1.0 / EXT), 0.0)
    wbd = jnp.broadcast_to(wg[:, None, :], (16, GS, 256)).reshape(128, 256)
    wbd = wbd * m2
    wf8 = jnp.dot(wbd, g[256 * gb:256 * gb + 256, :],
                  preferred_element_type=jnp.float32)
    for p in range(NKP):
      wfs_ref[P * p + GS * gb:P * p + GS * gb + GS, :] = wf8[8 * p:8 * p + 8, :]
  acc = jnp.zeros((P, C), dtype=jnp.float32)
  for p in range(NKP):
    acc = acc + jnp.dot(wfs_ref[P * p:P * p + P, :],
                        wcat_ref[p * C:(p + 1) * C, :],
                        preferred_element_type=jnp.float32)
  y_ref[...] = acc

  @pl.when(i == 0)
  def _():
    st_ref[...] = jnp.zeros_like(st_ref)
  st_ref[0:1, :] += jnp.sum(acc, axis=0, keepdims=True)
  st_ref[1:2, :] += jnp.sum(acc * acc, axis=0, keepdims=True)


def _tc_main(g, cen, kpa, m2, wcat, interpret=False):
  return pl.pallas_call(
      _tc_main_body,
      grid=(GRID,),
      in_specs=[
          pl.BlockSpec((PK, C), lambda i: (i, 0)),
          pl.BlockSpec((8 * NGB, 256), lambda i: (i, 0)),
          pl.BlockSpec((64, 256), lambda i: (0, 0)),
          pl.BlockSpec((128, 256), lambda i: (0, 0)),
          pl.BlockSpec((NKP * C, C), lambda i: (0, 0)),
      ],
      out_specs=[
          pl.BlockSpec((P, C), lambda i: (i, 0)),
          pl.BlockSpec((8, C), lambda i: (0, 0)),
      ],
      out_shape=[
          jax.ShapeDtypeStruct((NPAD, C), jnp.float32),
          jax.ShapeDtypeStruct((8, C), jnp.float32),
      ],
      scratch_shapes=[pltpu.VMEM((NKP * P, C), jnp.float32)],
      interpret=interpret,
  )(g, cen, kpa, m2, wcat)


def _tc_bn_body(y_ref, st_ref, gb_ref, o_ref):
  inv_n = 1.0 / N
  m = st_ref[0:1, :] * inv_n
  var = st_ref[1:2, :] * inv_n - m * m
  inv = lax.rsqrt(var + 1e-5)
  scale = gb_ref[0:1, :] * inv
  shift = gb_ref[1:2, :] - m * scale
  z = jnp.maximum(y_ref[...] * scale + shift, 0.0)
  o_ref[...] = z.T


BN_P = 1024


def _tc_bn(y, st, gb, interpret=False):
  return pl.pallas_call(
      _tc_bn_body,
      grid=(NPAD // BN_P,),
      in_specs=[
          pl.BlockSpec((BN_P, C), lambda i: (i, 0)),
          pl.BlockSpec((8, C), lambda i: (0, 0)),
          pl.BlockSpec((8, C), lambda i: (0, 0)),
      ],
      out_specs=pl.BlockSpec((C, BN_P), lambda i: (0, i)),
      out_shape=jax.ShapeDtypeStruct((C, NPAD), jnp.float32),
      interpret=interpret,
  )(y, st, gb)


def _prep(x, pxyz, pknn, kernel_points, weights, gamma, beta):
  feats = jnp.transpose(x[0, :, 0, :])
  xyzf = jnp.zeros((XYZF,), jnp.float32).at[:3 * N].set(
      pxyz[0].reshape(3 * N))
  idx = jnp.zeros((NKPAD,), jnp.int32).at[:NK].set(
      pknn[0].astype(jnp.int32).reshape(NK))
  kpe = jnp.concatenate(
      [kernel_points, jnp.array([[1e3, 0.0, 0.0]], jnp.float32)], axis=0)
  kpsq = jnp.sum(kpe * kpe, axis=1)
  kpa = jnp.concatenate(
      [jnp.broadcast_to(kpe[:, 0:1], (16, 256)),
       jnp.broadcast_to(kpe[:, 1:2], (16, 256)),
       jnp.broadcast_to(kpe[:, 2:3], (16, 256)),
       jnp.broadcast_to(kpsq[:, None], (16, 256))], axis=0)
  m2 = jnp.kron(jnp.ones((16, 1), jnp.float32),
                jnp.kron(jnp.eye(GS, dtype=jnp.float32),
                         jnp.ones((1, K), jnp.float32)))
  wcat = weights.reshape(NKP * C, C)
  gb = jnp.concatenate(
      [gamma[None, :], beta[None, :], jnp.zeros((6, C), jnp.float32)], axis=0)
  return feats, xyzf, idx, kpa, m2, wcat, gb


def kernel(x, pxyz, pknn, kernel_points, weights, gamma, beta):
  feats, xyzf, idx, kpa, m2, wcat, gb = _prep(
      x, pxyz, pknn, kernel_points, weights, gamma, beta)
  g, cen = _sc_gather(feats, xyzf, idx)
  y, st = _tc_main(g, cen, kpa, m2, wcat)
  out = _tc_bn(y, st, gb)
  return out[:, :N].reshape(1, C, 1, N)


# speedup vs baseline: 2.8239x; 2.8239x over previous
import functools

import jax
import jax.numpy as jnp
from jax import lax
from jax.experimental import pallas as pl
from jax.experimental.pallas import tpu as pltpu
from jax.experimental.pallas import tpu_sc as plsc

N = 10000
K = 32
NKP = 15
C = 128
EXT = 1.2
NK = N * K

NC = 2
NS = 16
NW = NC * NS
NKPAD = 327680
CH = 64
NCHUNKS = NKPAD // CH
N0 = 160
N1 = NCHUNKS // NS - N0
SPLIT0 = N0 * NS
NBUF = 2
NPT = 10240
XYZF = 30000
GS = 8
CROWS = NKPAD // (GS * K) * 8

P = 200
PK = P * K
GRID = N // P
NPAD = 10240


def _sc_gather(feats, xyzf, idx):
  mesh = plsc.VectorSubcoreMesh(core_axis_name="c", subcore_axis_name="s")

  @functools.partial(
      pl.kernel,
      out_type=[
          jax.ShapeDtypeStruct((NKPAD, C), jnp.float32),
          jax.ShapeDtypeStruct((CROWS, 256), jnp.float32),
      ],
      mesh=mesh,
      scratch_types=[
          pltpu.VMEM_SHARED((NPT, C), jnp.float32),
          pltpu.VMEM((XYZF,), jnp.float32),
          pltpu.VMEM((NBUF * CH,), jnp.int32),
          pltpu.VMEM((NBUF, CH, C), jnp.float32),
          pltpu.VMEM((NBUF, 8, 128), jnp.float32),
          pltpu.SemaphoreType.DMA((NBUF,)),
          pltpu.SemaphoreType.DMA((NBUF,)),
          pltpu.SemaphoreType.DMA((NBUF,)),
          pltpu.SemaphoreType.DMA((NBUF,)),
      ],
      compiler_params=pltpu.CompilerParams(needs_layout_passes=False,
                                           internal_scratch_in_bytes=1 << 16),
  )
  def sc_kernel(feats_hbm, xyz_hbm, idx_hbm, g_hbm, cen_hbm,
                feats_sh, xyz_t, idx_v, fbuf, cbuf,
                sem_i, sem_g, sem_fw, sem_cw):
    cid = lax.axis_index("c")
    sid = lax.axis_index("s")
    @pl.when(sid == 0)
    def _():
      pltpu.sync_copy(feats_hbm, feats_sh)
    pltpu.sync_copy(xyz_hbm, xyz_t)
    plsc.subcore_barrier()
    lanes16 = lax.iota(jnp.int32, 16)
    zero16 = jnp.zeros((16,), jnp.float32)

    def initrow(i, carry):
      b = i // 64
      r = (i // 8) % 8
      s = i % 8
      cbuf[b, r, pl.ds(s * 16, 16)] = zero16
      return carry

    lax.fori_loop(0, NBUF * 64, initrow, 0)

    def pipeline(start, iters):
      def chunk_base(i):
        return pl.multiple_of((start + i) * CH, CH)

      def idx_sl(i):
        return idx_v.at[pl.ds(pl.multiple_of((i % NBUF) * CH, CH), CH)]

      def idx_load(i):
        s = i % NBUF
        pltpu.async_copy(idx_hbm.at[pl.ds(chunk_base(i), CH)],
                         idx_sl(i), sem_i.at[s])

      def wait_idx(i):
        s = i % NBUF
        pltpu.make_async_copy(idx_hbm.at[pl.ds(chunk_base(i), CH)],
                              idx_sl(i), sem_i.at[s]).wait()

      def feats_out(i):
        s = i % NBUF
        pltpu.make_async_copy(feats_sh.at[idx_sl(i)],
                              fbuf.at[s], sem_g.at[s]).wait()
        pltpu.async_copy(fbuf.at[s], g_hbm.at[pl.ds(chunk_base(i), CH)],
                         sem_fw.at[s])

      def wait_feats_out(i):
        s = i % NBUF
        pltpu.make_async_copy(fbuf.at[s], g_hbm.at[pl.ds(chunk_base(i), CH)],
                              sem_fw.at[s]).wait()

      def _cen_dst(i):
        base = chunk_base(i - 1)
        row0 = pl.multiple_of((base // 256) * 8, 8)
        l0 = pl.multiple_of(lax.bitwise_and(base // 128, 1) * 128, 128)
        return cen_hbm.at[pl.ds(row0, 8), pl.ds(l0, 128)]

      def csl(i):
        return (i // 2) % NBUF

      def cen_out(i):
        pltpu.async_copy(cbuf.at[csl(i)], _cen_dst(i), sem_cw.at[csl(i)])

      def wait_cen_out(i):
        pltpu.make_async_copy(cbuf.at[csl(i)], _cen_dst(i),
                              sem_cw.at[csl(i)]).wait()

      idx_load(0)

      def body(i, carry):
        s = i % NBUF
        wait_idx(i)
        @pl.when(i >= NBUF)
        def _():
          wait_feats_out(i - NBUF)
        pltpu.async_copy(feats_sh.at[idx_sl(i)], fbuf.at[s], sem_g.at[s])
        @pl.when(i >= 1)
        def _():
          feats_out(i - 1)
        @pl.when(i + 1 < iters)
        def _():
          idx_load(i + 1)
        @pl.when(jnp.logical_and(i % 2 == 0, i >= 4))
        def _():
          wait_cen_out(i - 3)
        base = chunk_base(i)
        sc = csl(i)
        loff = lax.bitwise_and(i, 1) * 64
        for g in range(CH // 16):
          iv = idx_v[pl.ds(pl.multiple_of(s * CH, CH) + 16 * g, 16)]
          i3 = iv * 3
          r16 = base + 16 * g + lanes16
          q3 = jnp.minimum(lax.shift_right_logical(r16, 5) * 3, XYZF - 3)
          cx = plsc.load_gather(xyz_t, [i3]) - plsc.load_gather(xyz_t, [q3])
          cy = (plsc.load_gather(xyz_t, [i3 + 1])
                - plsc.load_gather(xyz_t, [q3 + 1]))
          cz = (plsc.load_gather(xyz_t, [i3 + 2])
                - plsc.load_gather(xyz_t, [q3 + 2]))
          cbuf[sc, 0, pl.ds(loff + 16 * g, 16)] = cx
          cbuf[sc, 1, pl.ds(loff + 16 * g, 16)] = cy
          cbuf[sc, 2, pl.ds(loff + 16 * g, 16)] = cz
        @pl.when(i % 2 == 1)
        def _():
          cen_out(i)
        return carry

      lax.fori_loop(0, iters, body, 0)
      feats_out(iters - 1)
      wait_feats_out(iters - 2)
      wait_feats_out(iters - 1)
      wait_cen_out(iters - 3)
      wait_cen_out(iters - 1)

    @pl.when(cid == 0)
    def _():
      pipeline(sid * N0, N0)

    @pl.when(cid != 0)
    def _():
      pipeline(SPLIT0 + sid * N1, N1)

  return sc_kernel(feats, xyzf, idx)


NGB = P // GS


def _tc_main_body(g_ref, cen_ref, kpa_ref, m2_ref, wcat_ref, y_ref, st_ref,
                  wfs_ref):
  i = pl.program_id(0)
  kpx = kpa_ref[0:16, :]
  kpy = kpa_ref[16:32, :]
  kpz = kpa_ref[32:48, :]
  kpq = kpa_ref[48:64, :]
  m2 = m2_ref[...]
  cen = cen_ref[...]
  g = g_ref[...]
  for gb in range(NGB):
    cx = cen[8 * gb + 0:8 * gb + 1, :]
    cy = cen[8 * gb + 1:8 * gb + 2, :]
    cz = cen[8 * gb + 2:8 * gb + 3, :]
    colsq = cx * cx + cy * cy + cz * cz
    d2 = kpq + colsq - 2.0 * (kpx * cx + kpy * cy + kpz * cz)
    wg = jnp.maximum(1.0 - jnp.sqrt(jnp.maximum(d2, 0.0)) * (1.0 / EXT), 0.0)
    wbd = jnp.broadcast_to(wg[:, None, :], (16, GS, 256)).reshape(128, 256)
    wbd = wbd * m2
    wf8 = jnp.dot(wbd, g[256 * gb:256 * gb + 256, :],
                  preferred_element_type=jnp.float32)
    for p in range(NKP):
      wfs_ref[P * p + GS * gb:P * p + GS * gb + GS, :] = wf8[8 * p:8 * p + 8, :]
  acc = jnp.zeros((P, C), dtype=jnp.float32)
  for p in range(NKP):
    acc = acc + jnp.dot(wfs_ref[P * p:P * p + P, :],
                        wcat_ref[p * C:(p + 1) * C, :],
                        preferred_element_type=jnp.float32)
  y_ref[...] = acc

  @pl.when(i == 0)
  def _():
    st_ref[...] = jnp.zeros_like(st_ref)
  st_ref[0:1, :] += jnp.sum(acc, axis=0, keepdims=True)
  st_ref[1:2, :] += jnp.sum(acc * acc, axis=0, keepdims=True)


def _tc_main(g, cen, kpa, m2, wcat, interpret=False):
  return pl.pallas_call(
      _tc_main_body,
      grid=(GRID,),
      in_specs=[
          pl.BlockSpec((PK, C), lambda i: (i, 0)),
          pl.BlockSpec((8 * NGB, 256), lambda i: (i, 0)),
          pl.BlockSpec((64, 256), lambda i: (0, 0)),
          pl.BlockSpec((128, 256), lambda i: (0, 0)),
          pl.BlockSpec((NKP * C, C), lambda i: (0, 0)),
      ],
      out_specs=[
          pl.BlockSpec((P, C), lambda i: (i, 0)),
          pl.BlockSpec((8, C), lambda i: (0, 0)),
      ],
      out_shape=[
          jax.ShapeDtypeStruct((NPAD, C), jnp.float32),
          jax.ShapeDtypeStruct((8, C), jnp.float32),
      ],
      scratch_shapes=[pltpu.VMEM((NKP * P, C), jnp.float32)],
      interpret=interpret,
  )(g, cen, kpa, m2, wcat)


def _tc_bn_body(y_ref, st_ref, gb_ref, o_ref):
  inv_n = 1.0 / N
  m = st_ref[0:1, :] * inv_n
  var = st_ref[1:2, :] * inv_n - m * m
  inv = lax.rsqrt(var + 1e-5)
  scale = gb_ref[0:1, :] * inv
  shift = gb_ref[1:2, :] - m * scale
  z = jnp.maximum(y_ref[...] * scale + shift, 0.0)
  o_ref[...] = z.T


BN_P = 1024


def _tc_bn(y, st, gb, interpret=False):
  return pl.pallas_call(
      _tc_bn_body,
      grid=(NPAD // BN_P,),
      in_specs=[
          pl.BlockSpec((BN_P, C), lambda i: (i, 0)),
          pl.BlockSpec((8, C), lambda i: (0, 0)),
          pl.BlockSpec((8, C), lambda i: (0, 0)),
      ],
      out_specs=pl.BlockSpec((C, BN_P), lambda i: (0, i)),
      out_shape=jax.ShapeDtypeStruct((C, NPAD), jnp.float32),
      interpret=interpret,
  )(y, st, gb)


def _prep(x, pxyz, pknn, kernel_points, weights, gamma, beta):
  feats = jnp.zeros((NPT, C), jnp.float32).at[:N, :].set(
      jnp.transpose(x[0, :, 0, :]))
  xyzf = jnp.zeros((XYZF,), jnp.float32).at[:3 * N].set(
      pxyz[0].reshape(3 * N))
  idx = jnp.zeros((NKPAD,), jnp.int32).at[:NK].set(
      pknn[0].astype(jnp.int32).reshape(NK))
  kpe = jnp.concatenate(
      [kernel_points, jnp.array([[1e3, 0.0, 0.0]], jnp.float32)], axis=0)
  kpsq = jnp.sum(kpe * kpe, axis=1)
  kpa = jnp.concatenate(
      [jnp.broadcast_to(kpe[:, 0:1], (16, 256)),
       jnp.broadcast_to(kpe[:, 1:2], (16, 256)),
       jnp.broadcast_to(kpe[:, 2:3], (16, 256)),
       jnp.broadcast_to(kpsq[:, None], (16, 256))], axis=0)
  m2 = jnp.kron(jnp.ones((16, 1), jnp.float32),
                jnp.kron(jnp.eye(GS, dtype=jnp.float32),
                         jnp.ones((1, K), jnp.float32)))
  wcat = weights.reshape(NKP * C, C)
  gb = jnp.concatenate(
      [gamma[None, :], beta[None, :], jnp.zeros((6, C), jnp.float32)], axis=0)
  return feats, xyzf, idx, kpa, m2, wcat, gb


def kernel(x, pxyz, pknn, kernel_points, weights, gamma, beta):
  feats, xyzf, idx, kpa, m2, wcat, gb = _prep(
      x, pxyz, pknn, kernel_points, weights, gamma, beta)
  g, cen = _sc_gather(feats, xyzf, idx)
  y, st = _tc_main(g, cen, kpa, m2, wcat)
  out = _tc_bn(y, st, gb)
  return out[:, :N].reshape(1, C, 1, N)
